# baseline (speedup 1.0000x reference)
import functools

import jax
import jax.numpy as jnp
from jax import lax
from jax.experimental import pallas as pl
from jax.experimental.pallas import tpu as pltpu
from jax.experimental.pallas import tpu_sc as plsc

_B = 4096
_L = 50
_EMBED_DIM = 64
_MASK_DIM = 16
_OUT_DIM = _EMBED_DIM + _MASK_DIM

_N = _B * _L
_NW = 32
_PER_W = _N // _NW
_C = 128
_CHUNKS = _PER_W // _C

_mesh = plsc.VectorSubcoreMesh(core_axis_name="c", subcore_axis_name="s")


@functools.partial(
    pl.kernel,
    mesh=_mesh,
    out_type=jax.ShapeDtypeStruct((_N, _OUT_DIM), jnp.float32),
    scratch_types=[
        pltpu.VMEM((_C,), jnp.int32),
        pltpu.VMEM((_C,), jnp.int32),
        pltpu.VMEM((_C, _EMBED_DIM), jnp.float32),
        pltpu.VMEM((_C, _MASK_DIM), jnp.float32),
        pltpu.SemaphoreType.DMA,
        pltpu.SemaphoreType.DMA,
    ],
    compiler_params=pltpu.CompilerParams(use_tc_tiling_on_sc=False),
)
def _embed_cat(sent_hbm, mask_hbm, word_hbm, mtab_hbm, out_hbm,
               sidx, midx, wrow, mrow, wsem, msem):
    wid = lax.axis_index("s") * 2 + lax.axis_index("c")
    wbase = wid * _PER_W

    def body(g, carry):
        base = wbase + g * _C
        pltpu.sync_copy(sent_hbm.at[pl.ds(base, _C)], sidx)
        pltpu.sync_copy(mask_hbm.at[pl.ds(base, _C)], midx)
        wcp = pltpu.async_copy(word_hbm.at[sidx], wrow, wsem)
        mcp = pltpu.async_copy(mtab_hbm.at[midx], mrow, msem)
        wcp.wait()
        mcp.wait()
        pltpu.sync_copy(wrow, out_hbm.at[pl.ds(base, _C), pl.ds(0, _EMBED_DIM)])
        pltpu.sync_copy(mrow, out_hbm.at[pl.ds(base, _C), pl.ds(_EMBED_DIM, _MASK_DIM)])
        return carry

    lax.fori_loop(0, _CHUNKS, body, 0)


def kernel(sent, mask, word_table, mask_table):
    s = sent.reshape(_N).astype(jnp.int32)
    m = mask.reshape(_N).astype(jnp.int32)
    out = _embed_cat(s, m, word_table, mask_table)
    return out.reshape(_B, _L, _OUT_DIM)


# speedup vs baseline: 1.0006x; 1.0006x over previous
import functools

import jax
import jax.numpy as jnp
from jax import lax
from jax.experimental import pallas as pl
from jax.experimental.pallas import tpu as pltpu
from jax.experimental.pallas import tpu_sc as plsc

_B = 4096
_L = 50
_EMBED_DIM = 64
_MASK_DIM = 16
_OUT_DIM = _EMBED_DIM + _MASK_DIM

_N = _B * _L
_NW = 32
_PER_W = _N // _NW
_C = 128
_CHUNKS = _PER_W // _C

_mesh = plsc.VectorSubcoreMesh(core_axis_name="c", subcore_axis_name="s")


@functools.partial(
    pl.kernel,
    mesh=_mesh,
    out_type=jax.ShapeDtypeStruct((_N, _OUT_DIM), jnp.float32),
    scratch_types=[
        pltpu.VMEM((_CHUNKS, _C), jnp.int32),
        pltpu.VMEM((_CHUNKS, _C), jnp.int32),
        pltpu.VMEM((_C, _EMBED_DIM), jnp.float32),
        pltpu.VMEM((_C, _EMBED_DIM), jnp.float32),
        pltpu.VMEM((_C, _MASK_DIM), jnp.float32),
        pltpu.VMEM((_C, _MASK_DIM), jnp.float32),
        pltpu.SemaphoreType.DMA,
        pltpu.SemaphoreType.DMA,
        pltpu.SemaphoreType.DMA,
        pltpu.SemaphoreType.DMA,
    ],
    compiler_params=pltpu.CompilerParams(use_tc_tiling_on_sc=False),
)
def _embed_cat(sent_hbm, mask_hbm, word_hbm, mtab_hbm, out_hbm,
               sidx, midx, w0, w1, m0, m1, gs0, gs1, ws0, ws1):
    wid = lax.axis_index("s") * 2 + lax.axis_index("c")
    wbase = wid * _PER_W

    pltpu.sync_copy(sent_hbm.at[wid], sidx)
    pltpu.sync_copy(mask_hbm.at[wid], midx)

    def fire_gather(g, wrow, mrow, sem):
        pltpu.async_copy(word_hbm.at[sidx.at[g]], wrow, sem)
        pltpu.async_copy(mtab_hbm.at[midx.at[g]], mrow, sem)

    def wait_gather(wrow, mrow, sem):
        pltpu.make_async_copy(word_hbm.at[sidx.at[0]], wrow, sem).wait()
        pltpu.make_async_copy(mtab_hbm.at[midx.at[0]], mrow, sem).wait()

    def fire_write(g, wrow, mrow, sem):
        rows = pl.ds(wbase + g * _C, _C)
        pltpu.async_copy(wrow, out_hbm.at[rows, pl.ds(0, _EMBED_DIM)], sem)
        pltpu.async_copy(mrow, out_hbm.at[rows, pl.ds(_EMBED_DIM, _MASK_DIM)], sem)

    def wait_write(wrow, mrow, sem):
        rows = pl.ds(wbase, _C)
        pltpu.make_async_copy(wrow, out_hbm.at[rows, pl.ds(0, _EMBED_DIM)], sem).wait()
        pltpu.make_async_copy(mrow, out_hbm.at[rows, pl.ds(_EMBED_DIM, _MASK_DIM)], sem).wait()

    fire_gather(0, w0, m0, gs0)

    def body(tt, carry):
        a = 2 * tt
        b = a + 1

        @pl.when(tt > 0)
        def _():
            wait_write(w1, m1, ws1)

        fire_gather(b, w1, m1, gs1)
        wait_gather(w0, m0, gs0)
        fire_write(a, w0, m0, ws0)
        wait_write(w0, m0, ws0)

        @pl.when(tt < _CHUNKS // 2 - 1)
        def _():
            fire_gather(a + 2, w0, m0, gs0)

        wait_gather(w1, m1, gs1)
        fire_write(b, w1, m1, ws1)
        return carry

    lax.fori_loop(0, _CHUNKS // 2, body, 0)
    wait_write(w1, m1, ws1)


def kernel(sent, mask, word_table, mask_table):
    s = sent.reshape(_NW, _CHUNKS, _C).astype(jnp.int32)
    m = mask.reshape(_NW, _CHUNKS, _C).astype(jnp.int32)
    out = _embed_cat(s, m, word_table, mask_table)
    return out.reshape(_B, _L, _OUT_DIM)


# speedup vs baseline: 2.1907x; 2.1894x over previous
import functools

import jax
import jax.numpy as jnp
from jax import lax
from jax.experimental import pallas as pl
from jax.experimental.pallas import tpu as pltpu
from jax.experimental.pallas import tpu_sc as plsc

_B = 4096
_L = 50
_EMBED_DIM = 64
_MASK_DIM = 16
_OUT_DIM = _EMBED_DIM + _MASK_DIM

_N = _B * _L
_NW = 32
_PER_W = _N // _NW
_C = 128
_CHUNKS = _PER_W // _C
_NBUF = 5
_D = 2
_T = _CHUNKS // _NBUF

_mesh = plsc.VectorSubcoreMesh(core_axis_name="c", subcore_axis_name="s")


@functools.partial(
    pl.kernel,
    mesh=_mesh,
    out_type=jax.ShapeDtypeStruct((_N, _OUT_DIM), jnp.float32),
    scratch_types=[
        pltpu.VMEM((_CHUNKS, _C), jnp.int32),
        pltpu.VMEM((_CHUNKS, _C), jnp.int32),
        pltpu.VMEM((2, _MASK_DIM), jnp.float32),
        pltpu.VMEM((_NBUF, _C, _EMBED_DIM), jnp.float32),
        pltpu.VMEM((_NBUF, _C, _MASK_DIM), jnp.float32),
    ]
    + [pltpu.SemaphoreType.DMA] * (3 * _NBUF),
    compiler_params=pltpu.CompilerParams(use_tc_tiling_on_sc=False),
)
def _embed_cat(sent_hbm, mask_hbm, word_hbm, mtab_hbm, out_hbm,
               sidx, midx, mtab, wbuf, mbuf, *sems):
    gsem = sems[:_NBUF]
    wsem = sems[_NBUF:2 * _NBUF]
    msem = sems[2 * _NBUF:]
    wid = lax.axis_index("s") * 2 + lax.axis_index("c")
    wbase = wid * _PER_W

    pltpu.sync_copy(sent_hbm.at[wid], sidx)
    pltpu.sync_copy(mask_hbm.at[wid], midx)
    pltpu.sync_copy(mtab_hbm, mtab)

    def fire_gather(c, s):
        pltpu.async_copy(word_hbm.at[sidx.at[c]], wbuf.at[s], gsem[s])

    def wait_gather(s):
        pltpu.make_async_copy(word_hbm.at[sidx.at[0]], wbuf.at[s],
                              gsem[s]).wait()

    def mask_fill(c, s):
        t0 = mtab[0, :]
        t1 = mtab[1, :]

        def body(v, carry):
            mv = midx[c, pl.ds(v * 16, 16)]
            base = v * 16
            for j in range(16):
                mbuf[s, base + j, :] = jnp.where(mv[j] == 0, t0, t1)
            return carry

        lax.fori_loop(0, _C // 16, body, 0)

    def fire_write(c, s):
        rows = pl.ds(wbase + c * _C, _C)
        pltpu.async_copy(wbuf.at[s], out_hbm.at[rows, pl.ds(0, _EMBED_DIM)],
                         wsem[s])
        pltpu.async_copy(mbuf.at[s],
                         out_hbm.at[rows, pl.ds(_EMBED_DIM, _MASK_DIM)],
                         msem[s])

    def wait_write(s):
        rows = pl.ds(wbase, _C)
        pltpu.make_async_copy(wbuf.at[s],
                              out_hbm.at[rows, pl.ds(0, _EMBED_DIM)],
                              wsem[s]).wait()
        pltpu.make_async_copy(mbuf.at[s],
                              out_hbm.at[rows, pl.ds(_EMBED_DIM, _MASK_DIM)],
                              msem[s]).wait()

    for c in range(_D):
        fire_gather(c, c)

    def body(t, carry):
        for j in range(_NBUF):
            c = t * _NBUF + j
            sn = (j + _D) % _NBUF

            if j < _NBUF - _D:
                @pl.when(t > 0)
                def _():
                    wait_write(sn)

                fire_gather(c + _D, sn)
            else:
                wait_write(sn)

                @pl.when(t < _T - 1)
                def _():
                    fire_gather(c + _D, sn)

            mask_fill(c, j)
            wait_gather(j)
            fire_write(c, j)
        return carry

    lax.fori_loop(0, _T, body, 0)

    for s in range(_D, _NBUF):
        wait_write(s)


def kernel(sent, mask, word_table, mask_table):
    s = sent.reshape(_NW, _CHUNKS, _C).astype(jnp.int32)
    m = mask.reshape(_NW, _CHUNKS, _C).astype(jnp.int32)
    out = _embed_cat(s, m, word_table, mask_table)
    return out.reshape(_B, _L, _OUT_DIM)


# speedup vs baseline: 2.1923x; 1.0007x over previous
import functools

import jax
import jax.numpy as jnp
from jax import lax
from jax.experimental import pallas as pl
from jax.experimental.pallas import tpu as pltpu
from jax.experimental.pallas import tpu_sc as plsc

_B = 4096
_L = 50
_EMBED_DIM = 64
_MASK_DIM = 16
_OUT_DIM = _EMBED_DIM + _MASK_DIM

_N = _B * _L
_NW = 32
_PER_W = _N // _NW
_C = 128
_CHUNKS = _PER_W // _C
_NBUF = 10
_D = 4
_T = _CHUNKS // _NBUF

_mesh = plsc.VectorSubcoreMesh(core_axis_name="c", subcore_axis_name="s")


@functools.partial(
    pl.kernel,
    mesh=_mesh,
    out_type=jax.ShapeDtypeStruct((_N, _OUT_DIM), jnp.float32),
    scratch_types=[
        pltpu.VMEM((_CHUNKS, _C), jnp.int32),
        pltpu.VMEM((_CHUNKS, _C), jnp.int32),
        pltpu.VMEM((2, _MASK_DIM), jnp.float32),
        pltpu.VMEM((_NBUF, _C, _EMBED_DIM), jnp.float32),
        pltpu.VMEM((_NBUF, _C, _MASK_DIM), jnp.float32),
    ]
    + [pltpu.SemaphoreType.DMA] * (3 * _NBUF),
    compiler_params=pltpu.CompilerParams(use_tc_tiling_on_sc=False),
)
def _embed_cat(sent_hbm, mask_hbm, word_hbm, mtab_hbm, out_hbm,
               sidx, midx, mtab, wbuf, mbuf, *sems):
    gsem = sems[:_NBUF]
    wsem = sems[_NBUF:2 * _NBUF]
    msem = sems[2 * _NBUF:]
    wid = lax.axis_index("s") * 2 + lax.axis_index("c")
    wbase = wid * _PER_W

    pltpu.sync_copy(sent_hbm.at[wid], sidx)
    pltpu.sync_copy(mask_hbm.at[wid], midx)
    pltpu.sync_copy(mtab_hbm, mtab)

    def fire_gather(c, s):
        pltpu.async_copy(word_hbm.at[sidx.at[c]], wbuf.at[s], gsem[s])

    def wait_gather(s):
        pltpu.make_async_copy(word_hbm.at[sidx.at[0]], wbuf.at[s],
                              gsem[s]).wait()

    def mask_fill(c, s):
        t0 = mtab[0, :]
        t1 = mtab[1, :]

        def body(v, carry):
            mv = midx[c, pl.ds(v * 16, 16)]
            base = v * 16
            for j in range(16):
                mbuf[s, base + j, :] = jnp.where(mv[j] == 0, t0, t1)
            return carry

        lax.fori_loop(0, _C // 16, body, 0)

    def fire_write(c, s):
        rows = pl.ds(wbase + c * _C, _C)
        pltpu.async_copy(wbuf.at[s], out_hbm.at[rows, pl.ds(0, _EMBED_DIM)],
                         wsem[s])
        pltpu.async_copy(mbuf.at[s],
                         out_hbm.at[rows, pl.ds(_EMBED_DIM, _MASK_DIM)],
                         msem[s])

    def wait_write(s):
        rows = pl.ds(wbase, _C)
        pltpu.make_async_copy(wbuf.at[s],
                              out_hbm.at[rows, pl.ds(0, _EMBED_DIM)],
                              wsem[s]).wait()
        pltpu.make_async_copy(mbuf.at[s],
                              out_hbm.at[rows, pl.ds(_EMBED_DIM, _MASK_DIM)],
                              msem[s]).wait()

    for c in range(_D):
        fire_gather(c, c)

    def body(t, carry):
        for j in range(_NBUF):
            c = t * _NBUF + j
            sn = (j + _D) % _NBUF

            if j < _NBUF - _D:
                @pl.when(t > 0)
                def _():
                    wait_write(sn)

                fire_gather(c + _D, sn)
            else:
                wait_write(sn)

                @pl.when(t < _T - 1)
                def _():
                    fire_gather(c + _D, sn)

            mask_fill(c, j)
            wait_gather(j)
            fire_write(c, j)
        return carry

    lax.fori_loop(0, _T, body, 0)

    for s in range(_D, _NBUF):
        wait_write(s)


def kernel(sent, mask, word_table, mask_table):
    s = sent.reshape(_NW, _CHUNKS, _C).astype(jnp.int32)
    m = mask.reshape(_NW, _CHUNKS, _C).astype(jnp.int32)
    out = _embed_cat(s, m, word_table, mask_table)
    return out.reshape(_B, _L, _OUT_DIM)
